# baseline (speedup 1.0000x reference)
import math
import functools

import jax
import jax.numpy as jnp
import numpy as np
from jax import lax
from jax.experimental import pallas as pl
from jax.experimental.pallas import tpu as pltpu
from jax.experimental.pallas import tpu_sc as plsc

VOCAB = 100000
HIDDEN = 768
MAX_LEN = 5000
EPS = 1e-12

NC = 2
NS = 16
NW = NC * NS
LANES = 16
NREG = HIDDEN // LANES


def _pe0_np(seq_len):
    pe = np.zeros((seq_len, HIDDEN), dtype=np.float32)
    position = np.arange(0, seq_len)[:, None].astype(np.float32)
    div_term = np.exp(
        np.arange(0, HIDDEN, 2, dtype=np.float32) * -(math.log(10000.0) / HIDDEN))
    pe[:, 0::2] = np.sin(position * div_term)
    pe[:, 1::2] = np.cos(position * div_term)
    return pe


def _rsqrt16(x_scalar):
    xv = jnp.full((LANES,), x_scalar, dtype=jnp.float32)
    iv = plsc.bitcast(xv, jnp.int32)
    magic = jnp.full((LANES,), np.int32(0x5F3759DF), dtype=jnp.int32)
    y = plsc.bitcast(magic - (iv >> 1), jnp.float32)
    half = jnp.full((LANES,), 0.5, dtype=jnp.float32) * xv
    for _ in range(3):
        y = y * (1.5 - half * y * y)
    return y


def _sc_body(ntok, chunk, word_hbm, ids_hbm, para_hbm, pe3_hbm, static_hbm,
             gamma_hbm, beta_hbm, out_hbm,
             idxw, idxp, wbuf, pbuf, sbuf, gv, bv, sem1, sem2):
    tok_w = ntok // NW
    nchunk = tok_w // chunk
    wid = lax.axis_index("s") * NC + lax.axis_index("c")

    pltpu.sync_copy(gamma_hbm, gv)
    pltpu.sync_copy(beta_hbm, bv)

    def chunk_body(ci, _):
        base = wid * tok_w + ci * chunk
        t0 = lax.rem(base, 512)
        pltpu.sync_copy(ids_hbm.at[pl.ds(base, chunk)], idxw)
        pltpu.sync_copy(para_hbm.at[pl.ds(base, chunk)], idxp)
        cp1 = pltpu.make_async_copy(word_hbm.at[idxw], wbuf, sem1)
        cp2 = pltpu.make_async_copy(pe3_hbm.at[idxp], pbuf, sem2)
        cp1.start()
        cp2.start()
        pltpu.sync_copy(static_hbm.at[pl.ds(t0, chunk)], sbuf)
        cp1.wait()
        cp2.wait()

        def tok_body(i, _):
            s1 = jnp.zeros((LANES,), jnp.float32)
            s2 = jnp.zeros((LANES,), jnp.float32)
            for j in range(NREG):
                sl = pl.ds(j * LANES, LANES)
                v = wbuf[i, sl] + pbuf[i, sl] + sbuf[i, sl]
                wbuf[i, sl] = v
                s1 = s1 + v
                s2 = s2 + v * v
            tot = jnp.sum(s1)
            mean = tot * (1.0 / HIDDEN)
            ms = jnp.sum(s2) * (1.0 / HIDDEN)
            var = ms - mean * mean
            inv = _rsqrt16(var + EPS)
            meanv = jnp.full((LANES,), mean, dtype=jnp.float32)
            for j in range(NREG):
                sl = pl.ds(j * LANES, LANES)
                wbuf[i, sl] = (wbuf[i, sl] - meanv) * inv * gv[sl] + bv[sl]
            return 0

        lax.fori_loop(0, chunk, tok_body, 0)
        pltpu.sync_copy(wbuf, out_hbm.at[pl.ds(base, chunk)])
        return 0

    lax.fori_loop(0, nchunk, chunk_body, 0)


def kernel(input_ids, tok_struct_vec, sent_struct_vec, word_emb, type_emb,
           ln_gamma, ln_beta):
    batch, seq = input_ids.shape
    ntok = batch * seq
    chunk = 32

    ids = input_ids.reshape(ntok).astype(jnp.int32)
    para = tok_struct_vec[:, :, 0].reshape(ntok).astype(jnp.int32)

    pe0 = jnp.asarray(_pe0_np(seq))
    pe3 = pe0 * 3.0
    static = pe0 + type_emb[0][None, :]

    mesh = plsc.VectorSubcoreMesh(
        core_axis_name="c", subcore_axis_name="s", num_cores=NC, num_subcores=NS)
    body = functools.partial(_sc_body, ntok, chunk)
    out = pl.kernel(
        body,
        out_type=jax.ShapeDtypeStruct((ntok, HIDDEN), jnp.float32),
        mesh=mesh,
        scratch_types=[
            pltpu.VMEM((chunk,), jnp.int32),
            pltpu.VMEM((chunk,), jnp.int32),
            pltpu.VMEM((chunk, HIDDEN), jnp.float32),
            pltpu.VMEM((chunk, HIDDEN), jnp.float32),
            pltpu.VMEM((chunk, HIDDEN), jnp.float32),
            pltpu.VMEM((HIDDEN,), jnp.float32),
            pltpu.VMEM((HIDDEN,), jnp.float32),
            pltpu.SemaphoreType.DMA,
            pltpu.SemaphoreType.DMA,
        ],
        compiler_params=pltpu.CompilerParams(needs_layout_passes=False),
    )(word_emb, ids, para, pe3, static, ln_gamma, ln_beta)
    return out.reshape(batch, seq, HIDDEN)


# speedup vs baseline: 1.8476x; 1.8476x over previous
import math
import functools

import jax
import jax.numpy as jnp
import numpy as np
from jax import lax
from jax.experimental import pallas as pl
from jax.experimental.pallas import tpu as pltpu
from jax.experimental.pallas import tpu_sc as plsc

VOCAB = 100000
HIDDEN = 768
MAX_LEN = 5000
EPS = 1e-12

NC = 2
NS = 16
NW = NC * NS
LANES = 16
NREG = HIDDEN // LANES
CHUNK = 32
NBUF = 2


def _pe0_np(seq_len):
    pe = np.zeros((seq_len, HIDDEN), dtype=np.float32)
    position = np.arange(0, seq_len)[:, None].astype(np.float32)
    div_term = np.exp(
        np.arange(0, HIDDEN, 2, dtype=np.float32) * -(math.log(10000.0) / HIDDEN))
    pe[:, 0::2] = np.sin(position * div_term)
    pe[:, 1::2] = np.cos(position * div_term)
    return pe


def _sc_gather_sum_body(ntok, word_hbm, ids_hbm, para_hbm, pe3_hbm, out_hbm,
                        idxw, idxp, wbufs, pbufs, semw, semp):
    tok_w = ntok // NW
    nchunk = tok_w // CHUNK
    wid = lax.axis_index("s") * NC + lax.axis_index("c")
    base0 = wid * tok_w

    def start(ci, slot):
        base = base0 + ci * CHUNK
        pltpu.sync_copy(ids_hbm.at[pl.ds(base, CHUNK)], idxw.at[slot])
        pltpu.sync_copy(para_hbm.at[pl.ds(base, CHUNK)], idxp.at[slot])
        pltpu.make_async_copy(word_hbm.at[idxw.at[slot]], wbufs.at[slot],
                              semw).start()
        pltpu.make_async_copy(pe3_hbm.at[idxp.at[slot]], pbufs.at[slot],
                              semp).start()

    for b in range(NBUF):
        start(b, b)

    def chunk_body(ci, _):
        slot = lax.rem(ci, NBUF)
        base = base0 + ci * CHUNK
        pltpu.make_async_copy(word_hbm.at[idxw.at[slot]], wbufs.at[slot],
                              semw).wait()
        pltpu.make_async_copy(pe3_hbm.at[idxp.at[slot]], pbufs.at[slot],
                              semp).wait()

        def tok_body(i, _):
            for j in range(NREG):
                sl = pl.ds(j * LANES, LANES)
                wbufs[slot, i, sl] = wbufs[slot, i, sl] + pbufs[slot, i, sl]
            return 0

        lax.fori_loop(0, CHUNK, tok_body, 0)
        pltpu.sync_copy(wbufs.at[slot], out_hbm.at[pl.ds(base, CHUNK)])

        @pl.when(ci + NBUF < nchunk)
        def _():
            start(ci + NBUF, slot)

        return 0

    lax.fori_loop(0, nchunk, chunk_body, 0)


def _tc_ln_body(wsum_ref, static_ref, g_ref, b_ref, out_ref):
    x = wsum_ref[...] + static_ref[...]
    mean = jnp.mean(x, axis=-1, keepdims=True)
    xc = x - mean
    var = jnp.mean(xc * xc, axis=-1, keepdims=True)
    y = xc * lax.rsqrt(var + EPS)
    out_ref[...] = y * g_ref[...] + b_ref[...]


def kernel(input_ids, tok_struct_vec, sent_struct_vec, word_emb, type_emb,
           ln_gamma, ln_beta):
    batch, seq = input_ids.shape
    ntok = batch * seq

    ids = input_ids.reshape(ntok).astype(jnp.int32)
    para = tok_struct_vec[:, :, 0].reshape(ntok).astype(jnp.int32)

    pe0 = jnp.asarray(_pe0_np(seq))
    pe3 = pe0 * 3.0
    static = pe0 + type_emb[0][None, :]

    mesh = plsc.VectorSubcoreMesh(
        core_axis_name="c", subcore_axis_name="s", num_cores=NC, num_subcores=NS)
    wsum = pl.kernel(
        functools.partial(_sc_gather_sum_body, ntok),
        out_type=jax.ShapeDtypeStruct((ntok, HIDDEN), jnp.float32),
        mesh=mesh,
        scratch_types=[
            pltpu.VMEM((NBUF, CHUNK), jnp.int32),
            pltpu.VMEM((NBUF, CHUNK), jnp.int32),
            pltpu.VMEM((NBUF, CHUNK, HIDDEN), jnp.float32),
            pltpu.VMEM((NBUF, CHUNK, HIDDEN), jnp.float32),
            pltpu.SemaphoreType.DMA,
            pltpu.SemaphoreType.DMA,
        ],
        compiler_params=pltpu.CompilerParams(needs_layout_passes=False),
    )(word_emb, ids, para, pe3)

    blk = 256
    grid = ntok // blk
    out = pl.pallas_call(
        _tc_ln_body,
        grid=(grid,),
        in_specs=[
            pl.BlockSpec((blk, HIDDEN), lambda i: (i, 0)),
            pl.BlockSpec((blk, HIDDEN), lambda i: (i % (seq // blk), 0)),
            pl.BlockSpec((1, HIDDEN), lambda i: (0, 0)),
            pl.BlockSpec((1, HIDDEN), lambda i: (0, 0)),
        ],
        out_specs=pl.BlockSpec((blk, HIDDEN), lambda i: (i, 0)),
        out_shape=jax.ShapeDtypeStruct((ntok, HIDDEN), jnp.float32),
    )(wsum, static, ln_gamma.reshape(1, HIDDEN), ln_beta.reshape(1, HIDDEN))

    return out.reshape(batch, seq, HIDDEN)


# speedup vs baseline: 2.1296x; 1.1526x over previous
import math
import functools

import jax
import jax.numpy as jnp
import numpy as np
from jax import lax
from jax.experimental import pallas as pl
from jax.experimental.pallas import tpu as pltpu
from jax.experimental.pallas import tpu_sc as plsc

VOCAB = 100000
HIDDEN = 768
MAX_LEN = 5000
EPS = 1e-12

NC = 2
NS = 16
NW = NC * NS
LANES = 16
NREG = HIDDEN // LANES
CHUNK = 32
NBUF = 2


def _pe0_np(seq_len):
    pe = np.zeros((seq_len, HIDDEN), dtype=np.float32)
    position = np.arange(0, seq_len)[:, None].astype(np.float32)
    div_term = np.exp(
        np.arange(0, HIDDEN, 2, dtype=np.float32) * -(math.log(10000.0) / HIDDEN))
    pe[:, 0::2] = np.sin(position * div_term)
    pe[:, 1::2] = np.cos(position * div_term)
    return pe


def _sc_gather_sum_body(ntok, word_hbm, ids_hbm, para_hbm, pe3_hbm, out_hbm,
                        idxw, idxp, wbufs, pbufs, semw, semp):
    tok_w = ntok // NW
    nchunk = tok_w // CHUNK
    wid = lax.axis_index("s") * NC + lax.axis_index("c")
    base0 = wid * tok_w

    def start(ci, slot):
        base = base0 + ci * CHUNK
        pltpu.sync_copy(ids_hbm.at[pl.ds(base, CHUNK)], idxw.at[slot])
        pltpu.sync_copy(para_hbm.at[pl.ds(base, CHUNK)], idxp.at[slot])
        pltpu.make_async_copy(word_hbm.at[idxw.at[slot]], wbufs.at[slot],
                              semw).start()
        pltpu.make_async_copy(pe3_hbm.at[idxp.at[slot]], pbufs.at[slot],
                              semp).start()

    for b in range(NBUF):
        start(b, b)

    def chunk_body(ci, _):
        slot = lax.rem(ci, NBUF)
        base = base0 + ci * CHUNK
        pltpu.make_async_copy(word_hbm.at[idxw.at[slot]], wbufs.at[slot],
                              semw).wait()
        pltpu.make_async_copy(pe3_hbm.at[idxp.at[slot]], pbufs.at[slot],
                              semp).wait()

        def tok_body(i, _):
            for j in range(NREG):
                sl = pl.ds(j * LANES, LANES)
                wbufs[slot, i, sl] = wbufs[slot, i, sl] + pbufs[slot, i, sl]
            return 0

        lax.fori_loop(0, CHUNK, tok_body, 0)
        pltpu.sync_copy(wbufs.at[slot], out_hbm.at[pl.ds(base, CHUNK)])

        @pl.when(ci + NBUF < nchunk)
        def _():
            start(ci + NBUF, slot)

        return 0

    lax.fori_loop(0, nchunk, chunk_body, 0)


def _tc_ln_body(wsum_ref, static_ref, g_ref, b_ref, out_ref):
    x = wsum_ref[...] + static_ref[...]
    mean = jnp.mean(x, axis=-1, keepdims=True)
    xc = x - mean
    var = jnp.mean(xc * xc, axis=-1, keepdims=True)
    y = xc * lax.rsqrt(var + EPS)
    out_ref[...] = y * g_ref[...] + b_ref[...]


def kernel(input_ids, tok_struct_vec, sent_struct_vec, word_emb, type_emb,
           ln_gamma, ln_beta):
    batch, seq = input_ids.shape
    ntok = batch * seq

    ids = input_ids.reshape(ntok).astype(jnp.int32)
    para = tok_struct_vec[:, :, 0].reshape(ntok).astype(jnp.int32)

    pe0 = jnp.asarray(_pe0_np(seq))
    pe3 = pe0 * 3.0
    static = pe0 + type_emb[0][None, :]

    mesh = plsc.VectorSubcoreMesh(
        core_axis_name="c", subcore_axis_name="s", num_cores=NC, num_subcores=NS)
    wsum = pl.kernel(
        functools.partial(_sc_gather_sum_body, ntok),
        out_type=jax.ShapeDtypeStruct((ntok, HIDDEN), jnp.float32),
        mesh=mesh,
        scratch_types=[
            pltpu.VMEM((NBUF, CHUNK), jnp.int32),
            pltpu.VMEM((NBUF, CHUNK), jnp.int32),
            pltpu.VMEM((NBUF, CHUNK, HIDDEN), jnp.float32),
            pltpu.VMEM((NBUF, CHUNK, HIDDEN), jnp.float32),
            pltpu.SemaphoreType.DMA,
            pltpu.SemaphoreType.DMA,
        ],
        compiler_params=pltpu.CompilerParams(needs_layout_passes=False),
    )(word_emb, ids, para, pe3)

    blk = seq
    grid = ntok // blk
    out = pl.pallas_call(
        _tc_ln_body,
        grid=(grid,),
        in_specs=[
            pl.BlockSpec((blk, HIDDEN), lambda i: (i, 0)),
            pl.BlockSpec((blk, HIDDEN), lambda i: (0, 0)),
            pl.BlockSpec((1, HIDDEN), lambda i: (0, 0)),
            pl.BlockSpec((1, HIDDEN), lambda i: (0, 0)),
        ],
        out_specs=pl.BlockSpec((blk, HIDDEN), lambda i: (i, 0)),
        out_shape=jax.ShapeDtypeStruct((ntok, HIDDEN), jnp.float32),
    )(wsum, static, ln_gamma.reshape(1, HIDDEN), ln_beta.reshape(1, HIDDEN))

    return out.reshape(batch, seq, HIDDEN)
